# Initial kernel scaffold; baseline (speedup 1.0000x reference)
import jax
import jax.numpy as jnp
from jax.experimental import pallas as pl


def kernel(x_leaves):
    raise NotImplementedError("write your pallas kernel here")


# speedup vs baseline: 2.9979x; 2.9979x over previous
import jax
import jax.numpy as jnp
from jax.experimental import pallas as pl

_N = 4096
_ROWS_PER_BLOCK = 256


def _outer_sum_kernel(row_ref, col_ref, out_ref):
    row = row_ref[...].reshape(_ROWS_PER_BLOCK, 1)
    out_ref[...] = row + (col_ref[...] + 1.0)


def kernel(x_leaves):
    n = x_leaves.shape[1]
    grid = (n // _ROWS_PER_BLOCK,)
    out = pl.pallas_call(
        _outer_sum_kernel,
        grid=grid,
        in_specs=[
            pl.BlockSpec((1, _ROWS_PER_BLOCK), lambda i: (0, i)),
            pl.BlockSpec((1, n), lambda i: (0, 0)),
        ],
        out_specs=pl.BlockSpec((_ROWS_PER_BLOCK, n), lambda i: (i, 0)),
        out_shape=jax.ShapeDtypeStruct((n, n), jnp.float32),
    )(x_leaves, x_leaves)
    return out.reshape(1, n * n)
